# baseline (speedup 1.0000x reference)
import jax
import jax.numpy as jnp
from jax.experimental import pallas as pl
from jax.experimental.pallas import tpu as pltpu


def _make_gcn_kernel(B, N, F, H1, H2):
    def body(length_ref, x_ref, adj_ref, w1_ref, b1_ref, w2_ref, b2_ref,
             wlin_ref, blin_ref, out_ref, abuf, s2buf, sems):
        t = pl.program_id(0)
        cur = jax.lax.rem(t, 3)
        prv = jax.lax.rem(t + 2, 3)
        nxt = jax.lax.rem(t + 1, 3)
        c2 = jax.lax.rem(t, 2)
        o2 = jax.lax.rem(t + 1, 2)
        nchunk = 4
        rows = N // nchunk

        def copy(b_idx, slot, k):
            return pltpu.make_async_copy(
                adj_ref.at[b_idx, pl.ds(k * rows, rows), :],
                abuf.at[slot, pl.ds(k * rows, rows), :],
                sems.at[slot, k])

        @pl.when(t == 0)
        def _():
            for k in range(nchunk):
                copy(0, 0, k).start()

        @pl.when(t + 1 < B)
        def _():
            for k in range(nchunk):
                copy(t + 1, nxt, k).start()

        @pl.when(t > 0)
        def _():
            h2 = jnp.maximum(
                jnp.dot(abuf[prv], s2buf[o2],
                        preferred_element_type=jnp.float32) + b2_ref[0], 0.0)
            length = length_ref[t - 1]
            mask = jax.lax.broadcasted_iota(jnp.int32, (N, 1), 0) < length
            pooled = (jnp.sum(jnp.where(mask, h2, 0.0), axis=0, keepdims=True)
                      / length.astype(jnp.float32))
            out_ref[0] = jnp.dot(pooled, wlin_ref[...]) + blin_ref[0]

        @pl.when(t == 0)
        def _():
            s1 = jnp.dot(x_ref[0], w1_ref[...],
                         preferred_element_type=jnp.float32)
            for k in range(nchunk):
                copy(0, 0, k).wait()
                rk = pl.ds(k * rows, rows)
                h_k = jnp.maximum(
                    jnp.dot(abuf[0, rk, :], s1,
                            preferred_element_type=jnp.float32) + b1_ref[0],
                    0.0)
                s2buf[0, rk, :] = jnp.dot(h_k, w2_ref[...],
                                          preferred_element_type=jnp.float32)

        @pl.when(jnp.logical_and(t > 0, t < B))
        def _():
            for k in range(nchunk):
                copy(t, cur, k).wait()
            s1 = jnp.dot(x_ref[0], w1_ref[...],
                         preferred_element_type=jnp.float32)
            h = jnp.maximum(
                jnp.dot(abuf[cur], s1,
                        preferred_element_type=jnp.float32) + b1_ref[0], 0.0)
            s2buf[c2] = jnp.dot(h, w2_ref[...],
                                preferred_element_type=jnp.float32)

    return body


def kernel(x, adj, length, W1, b1, W2, b2, Wlin, blin):
    B, N, F = x.shape
    H1 = W1.shape[1]
    H2 = W2.shape[1]

    grid_spec = pltpu.PrefetchScalarGridSpec(
        num_scalar_prefetch=1,
        grid=(B + 1,),
        in_specs=[
            pl.BlockSpec((1, N, F), lambda t, L: (jnp.minimum(t, B - 1), 0, 0)),
            pl.BlockSpec(memory_space=pltpu.MemorySpace.HBM),
            pl.BlockSpec((F, H1), lambda t, L: (0, 0)),
            pl.BlockSpec((1, H1), lambda t, L: (0, 0)),
            pl.BlockSpec((H1, H2), lambda t, L: (0, 0)),
            pl.BlockSpec((1, H2), lambda t, L: (0, 0)),
            pl.BlockSpec((H2, 1), lambda t, L: (0, 0)),
            pl.BlockSpec((1, 1), lambda t, L: (0, 0)),
        ],
        out_specs=pl.BlockSpec((1, 1, 1),
                               lambda t, L: (jnp.maximum(t - 1, 0), 0, 0)),
        scratch_shapes=[
            pltpu.VMEM((3, N, N), jnp.float32),
            pltpu.VMEM((2, N, H2), jnp.float32),
            pltpu.SemaphoreType.DMA((3, 4)),
        ],
    )

    out = pl.pallas_call(
        _make_gcn_kernel(B, N, F, H1, H2),
        grid_spec=grid_spec,
        out_shape=jax.ShapeDtypeStruct((B, 1, 1), jnp.float32),
    )(length, x, adj, W1, b1.reshape(1, H1), W2, b2.reshape(1, H2),
      Wlin, blin.reshape(1, 1))
    return out.reshape(B, 1)


# speedup vs baseline: 1.0050x; 1.0050x over previous
import jax
import jax.numpy as jnp
from jax.experimental import pallas as pl
from jax.experimental.pallas import tpu as pltpu


def _make_gcn_kernel(B, N, F, H1, H2):
    def body(length_ref, x_ref, adj_ref, w1_ref, b1_ref, w2_ref, b2_ref,
             wlin_ref, blin_ref, out_ref, abuf, s2buf, pacc, sems):
        t = pl.program_id(0)
        cur = jax.lax.rem(t, 3)
        prv = jax.lax.rem(t + 2, 3)
        nxt = jax.lax.rem(t + 1, 3)
        c2 = jax.lax.rem(t, 2)
        o2 = jax.lax.rem(t + 1, 2)
        nchunk = 4
        rows = N // nchunk

        def copy(b_idx, slot, k):
            return pltpu.make_async_copy(
                adj_ref.at[b_idx, pl.ds(k * rows, rows), :],
                abuf.at[slot, pl.ds(k * rows, rows), :],
                sems.at[slot, k])

        @pl.when(t == 0)
        def _():
            for k in range(nchunk):
                copy(0, 0, k).start()

        @pl.when(t + 1 < B)
        def _():
            for k in range(nchunk):
                copy(t + 1, nxt, k).start()

        @pl.when(t > 0)
        def _():
            length = length_ref[t - 1]
            for k in range(nchunk):
                rk = pl.ds(k * rows, rows)

                @pl.when(k * rows < length)
                def _():
                    h2_k = jnp.maximum(
                        jnp.dot(abuf[prv, rk, :], s2buf[o2],
                                preferred_element_type=jnp.float32)
                        + b2_ref[0], 0.0)
                    mask = (jax.lax.broadcasted_iota(jnp.int32, (rows, 1), 0)
                            < length - k * rows)
                    pacc[k, :] = jnp.sum(jnp.where(mask, h2_k, 0.0), axis=0)

                @pl.when(k * rows >= length)
                def _():
                    pacc[k, :] = jnp.zeros((H2,), jnp.float32)

            pooled = (jnp.sum(pacc[...], axis=0, keepdims=True)
                      / length.astype(jnp.float32))
            out_ref[0] = jnp.dot(pooled, wlin_ref[...]) + blin_ref[0]

        @pl.when(t == 0)
        def _():
            s1 = jnp.dot(x_ref[0], w1_ref[...],
                         preferred_element_type=jnp.float32)
            for k in range(nchunk):
                copy(0, 0, k).wait()
                rk = pl.ds(k * rows, rows)
                h_k = jnp.maximum(
                    jnp.dot(abuf[0, rk, :], s1,
                            preferred_element_type=jnp.float32) + b1_ref[0],
                    0.0)
                s2buf[0, rk, :] = jnp.dot(h_k, w2_ref[...],
                                          preferred_element_type=jnp.float32)

        @pl.when(jnp.logical_and(t > 0, t < B))
        def _():
            for k in range(nchunk):
                copy(t, cur, k).wait()
            s1 = jnp.dot(x_ref[0], w1_ref[...],
                         preferred_element_type=jnp.float32)
            h = jnp.maximum(
                jnp.dot(abuf[cur], s1,
                        preferred_element_type=jnp.float32) + b1_ref[0], 0.0)
            s2buf[c2] = jnp.dot(h, w2_ref[...],
                                preferred_element_type=jnp.float32)

    return body


def kernel(x, adj, length, W1, b1, W2, b2, Wlin, blin):
    B, N, F = x.shape
    H1 = W1.shape[1]
    H2 = W2.shape[1]

    grid_spec = pltpu.PrefetchScalarGridSpec(
        num_scalar_prefetch=1,
        grid=(B + 1,),
        in_specs=[
            pl.BlockSpec((1, N, F), lambda t, L: (jnp.minimum(t, B - 1), 0, 0)),
            pl.BlockSpec(memory_space=pltpu.MemorySpace.HBM),
            pl.BlockSpec((F, H1), lambda t, L: (0, 0)),
            pl.BlockSpec((1, H1), lambda t, L: (0, 0)),
            pl.BlockSpec((H1, H2), lambda t, L: (0, 0)),
            pl.BlockSpec((1, H2), lambda t, L: (0, 0)),
            pl.BlockSpec((H2, 1), lambda t, L: (0, 0)),
            pl.BlockSpec((1, 1), lambda t, L: (0, 0)),
        ],
        out_specs=pl.BlockSpec((1, 1, 1),
                               lambda t, L: (jnp.maximum(t - 1, 0), 0, 0)),
        scratch_shapes=[
            pltpu.VMEM((3, N, N), jnp.float32),
            pltpu.VMEM((2, N, H2), jnp.float32),
            pltpu.VMEM((4, H2), jnp.float32),
            pltpu.SemaphoreType.DMA((3, 4)),
        ],
    )

    out = pl.pallas_call(
        _make_gcn_kernel(B, N, F, H1, H2),
        grid_spec=grid_spec,
        out_shape=jax.ShapeDtypeStruct((B, 1, 1), jnp.float32),
    )(length, x, adj, W1, b1.reshape(1, H1), W2, b2.reshape(1, H2),
      Wlin, blin.reshape(1, 1))
    return out.reshape(B, 1)
